# baseline (speedup 1.0000x reference)
import jax
import jax.numpy as jnp
from jax import lax
from jax.experimental import pallas as pl
from jax.experimental.pallas import tpu as pltpu
from jax.experimental.pallas import tpu_sc as plsc

C, G, S, L, B = 4, 8192, 8, 4, 16
INFER_STEP = 2
GAMMA = 0.01
INVG = float(1.0 / GAMMA)

NC, NS = 2, 16
NW = NC * NS
NG = 16
RPC = NG * S * L
IDX_ROWS = RPC // 128
RING = 4


def _make_sc(nclause):
    nchunk = nclause * G // (NW * NG)

    def body(tab, idx_hbm, m_out, s_out, idx_v, rows_v, mbuf, sbuf, *sems):
        rows_sems = sems[0:4]
        idx_sems = sems[4:8]
        out_sems = sems[8:12]

        wid = lax.axis_index("s") * NC + lax.axis_index("c")
        base = wid * nchunk

        def idx_slice(off):
            q = base + jnp.minimum(off, nchunk - 1)
            return idx_hbm.at[pl.ds(q * IDX_ROWS, IDX_ROWS)]

        def start_idx(off, p):
            pltpu.async_copy(idx_slice(off), idx_v.at[p], idx_sems[p])

        def wait_idx(p):
            pltpu.make_async_copy(idx_slice(0), idx_v.at[p], idx_sems[p]).wait()

        def start_gathers(p):
            for j in range(IDX_ROWS):
                pltpu.async_copy(tab.at[idx_v.at[p, j]], rows_v.at[p, j],
                                 rows_sems[p])

        def wait_gathers(p):
            for j in range(IDX_ROWS):
                pltpu.make_async_copy(tab.at[idx_v.at[p, j]], rows_v.at[p, j],
                                      rows_sems[p]).wait()

        def out_slices(off):
            q = base + off
            return (m_out.at[pl.ds(q * NG * B, NG * B)],
                    s_out.at[pl.ds(q * NG * B, NG * B)])

        def start_out(off, p):
            mo, so = out_slices(off)
            pltpu.async_copy(mbuf.at[p], mo, out_sems[p])
            pltpu.async_copy(sbuf.at[p], so, out_sems[p])

        def wait_out(p):
            mo, so = out_slices(0)
            pltpu.make_async_copy(mbuf.at[p], mo, out_sems[p]).wait()
            pltpu.make_async_copy(sbuf.at[p], so, out_sems[p]).wait()

        def compute(p):
            for gl in range(NG):
                r0 = gl * S * L
                ps = []
                for s in range(S):
                    k = r0 + s * L
                    v = rows_v[p, k // 128, k % 128]
                    for l in range(1, L):
                        v = v * rows_v[p, (k + l) // 128, (k + l) % 128]
                    ps.append(v)
                m = ps[0]
                for s in range(1, S):
                    m = jnp.maximum(m, ps[s])
                acc = jnp.exp((ps[0] - m) * INVG)
                for s in range(1, S):
                    acc = acc + jnp.exp((ps[s] - m) * INVG)
                mbuf[p, pl.ds(gl * B, B)] = m
                sbuf[p, pl.ds(gl * B, B)] = acc

        pltpu.sync_copy(idx_slice(0), idx_v.at[0])
        start_gathers(0)
        start_idx(1, 1)
        wait_idx(1)
        start_gathers(1)
        start_idx(2, 2)

        def outer(i, carry):
            off0 = i * RING
            for u in range(RING):
                off = off0 + u
                p = u
                p2 = (u + 2) % RING
                p3 = (u + 3) % RING
                wait_idx(p2)
                start_gathers(p2)
                wait_gathers(p)
                start_idx(off + 3, p3)
                @pl.when(off >= RING)
                def _():
                    wait_out(p)
                compute(p)
                start_out(off, p)
            return carry

        lax.fori_loop(0, nchunk // RING, outer, 0)

        wait_gathers(0)
        wait_gathers(1)
        wait_idx(2)
        for p in range(RING):
            wait_out(p)

    return pl.kernel(
        body,
        out_type=(
            jax.ShapeDtypeStruct((nclause * G * B,), jnp.float32),
            jax.ShapeDtypeStruct((nclause * G * B,), jnp.float32),
        ),
        mesh=plsc.VectorSubcoreMesh(
            core_axis_name="c", subcore_axis_name="s",
            num_cores=NC, num_subcores=NS,
        ),
        scratch_types=[
            pltpu.VMEM((RING, IDX_ROWS, 128), jnp.int32),
            pltpu.VMEM((RING, IDX_ROWS, 128, B), jnp.float32),
            pltpu.VMEM((RING, NG * B), jnp.float32),
            pltpu.VMEM((RING, NG * B), jnp.float32),
        ] + [pltpu.SemaphoreType.DMA] * 12,
        compiler_params=pltpu.CompilerParams(use_tc_tiling_on_sc=False),
    )


_sc_clause = _make_sc(1)
_sc_all = _make_sc(C)


def _tc_body1(R_ref, *refs):
    ms = refs[0:C]
    ss = refs[C:2 * C]
    out_ref = refs[2 * C]
    us = []
    M = None
    for c in range(C):
        t = ms[c][:] + GAMMA * jnp.log(ss[c][:])
        mx = jnp.max(t)
        r = t / jnp.maximum(mx, 1.0)
        Rc = R_ref[:]
        mm = jnp.maximum(Rc, r)
        u = mm + GAMMA * jnp.log(
            jnp.exp((Rc - mm) * INVG) + jnp.exp((r - mm) * INVG)
        )
        us.append(u)
        uM = jnp.max(u)
        M = uM if M is None else jnp.maximum(M, uM)
    scale = 1.0 / jnp.maximum(M, 1.0)
    for c in range(C):
        out_ref[c] = us[c] * scale


def _tc_body2(R_ref, m_ref, s_ref, out_ref):
    us = []
    M = None
    for c in range(C):
        t = m_ref[c] + GAMMA * jnp.log(s_ref[c])
        mx = jnp.max(t)
        r = t / jnp.maximum(mx, 1.0)
        Rc = R_ref[c]
        mm = jnp.maximum(Rc, r)
        u = mm + GAMMA * jnp.log(
            jnp.exp((Rc - mm) * INVG) + jnp.exp((r - mm) * INVG)
        )
        us.append(u)
        uM = jnp.max(u)
        M = uM if M is None else jnp.maximum(M, uM)
    scale = 1.0 / jnp.maximum(M, 1.0)
    for c in range(C):
        out_ref[c] = us[c] * scale


_tc_combine1 = pl.pallas_call(
    _tc_body1,
    out_shape=jax.ShapeDtypeStruct((C, G * B), jnp.float32),
)
_tc_combine2 = pl.pallas_call(
    _tc_body2,
    out_shape=jax.ShapeDtypeStruct((C, G * B), jnp.float32),
)


def kernel(x, I):
    xT = x.T
    iflats = [I[c].reshape(G * S * L // 128, 128) for c in range(C)]
    parts = [_sc_clause(xT, iflats[c]) for c in range(C)]
    ms = [p[0] for p in parts]
    ss = [p[1] for p in parts]
    Rt = _tc_combine1(xT.reshape(G * B), *ms, *ss)
    iadj = jnp.stack([iflats[c] + c * G for c in range(C)]).reshape(-1, 128)
    for _ in range(INFER_STEP - 1):
        m, s = _sc_all(Rt.reshape(C * G, B), iadj)
        Rt = _tc_combine2(Rt, m.reshape(C, G * B), s.reshape(C, G * B))
    return Rt.reshape(C, G, B).transpose(0, 2, 1)


# speedup vs baseline: 1.1964x; 1.1964x over previous
import jax
import jax.numpy as jnp
from jax import lax
from jax.experimental import pallas as pl
from jax.experimental.pallas import tpu as pltpu
from jax.experimental.pallas import tpu_sc as plsc

C, G, S, L, B = 4, 8192, 8, 4, 16
INFER_STEP = 2
GAMMA = 0.01
INVG = float(1.0 / GAMMA)

NC, NS = 2, 16
NW = NC * NS
NG = 16
RPC = NG * S * L
IDX_ROWS = RPC // 128
RING = 4


def _make_sc(nclause):
    nchunk = nclause * G // (NW * NG)

    def body(tab, idx_hbm, m_out, s_out, idx_v, rows_v, mbuf, sbuf, *sems):
        rows_sems = sems[0:4]
        idx_sems = sems[4:8]
        out_sems = sems[8:12]

        wid = lax.axis_index("s") * NC + lax.axis_index("c")
        base = wid * nchunk

        def idx_slice(off):
            q = base + jnp.minimum(off, nchunk - 1)
            return idx_hbm.at[pl.ds(q * IDX_ROWS, IDX_ROWS)]

        def start_idx(off, p):
            pltpu.async_copy(idx_slice(off), idx_v.at[p], idx_sems[p])

        def wait_idx(p):
            pltpu.make_async_copy(idx_slice(0), idx_v.at[p], idx_sems[p]).wait()

        def start_gathers(p):
            for j in range(IDX_ROWS):
                pltpu.async_copy(tab.at[idx_v.at[p, j]], rows_v.at[p, j],
                                 rows_sems[p])

        def wait_gathers(p):
            for j in range(IDX_ROWS):
                pltpu.make_async_copy(tab.at[idx_v.at[p, j]], rows_v.at[p, j],
                                      rows_sems[p]).wait()

        def out_slices(off):
            q = base + off
            return (m_out.at[pl.ds(q * NG * B, NG * B)],
                    s_out.at[pl.ds(q * NG * B, NG * B)])

        def start_out(off, p):
            mo, so = out_slices(off)
            pltpu.async_copy(mbuf.at[p], mo, out_sems[p])
            pltpu.async_copy(sbuf.at[p], so, out_sems[p])

        def wait_out(p):
            mo, so = out_slices(0)
            pltpu.make_async_copy(mbuf.at[p], mo, out_sems[p]).wait()
            pltpu.make_async_copy(sbuf.at[p], so, out_sems[p]).wait()

        def compute(p):
            for gl in range(NG):
                r0 = gl * S * L
                ps = []
                for s in range(S):
                    k = r0 + s * L
                    v = rows_v[p, k // 128, k % 128]
                    for l in range(1, L):
                        v = v * rows_v[p, (k + l) // 128, (k + l) % 128]
                    ps.append(v)
                m = ps[0]
                for s in range(1, S):
                    m = jnp.maximum(m, ps[s])
                acc = jnp.exp((ps[0] - m) * INVG)
                for s in range(1, S):
                    acc = acc + jnp.exp((ps[s] - m) * INVG)
                mbuf[p, pl.ds(gl * B, B)] = m
                sbuf[p, pl.ds(gl * B, B)] = acc

        pltpu.sync_copy(idx_slice(0), idx_v.at[0])
        start_gathers(0)
        start_idx(1, 1)
        wait_idx(1)
        start_gathers(1)
        start_idx(2, 2)

        def outer(i, carry):
            off0 = i * RING
            for u in range(RING):
                off = off0 + u
                p = u
                p2 = (u + 2) % RING
                p3 = (u + 3) % RING
                wait_idx(p2)
                start_gathers(p2)
                wait_gathers(p)
                start_idx(off + 3, p3)
                @pl.when(off >= RING)
                def _():
                    wait_out(p)
                compute(p)
                start_out(off, p)
            return carry

        lax.fori_loop(0, nchunk // RING, outer, 0)

        wait_gathers(0)
        wait_gathers(1)
        wait_idx(2)
        for p in range(RING):
            wait_out(p)

    return pl.kernel(
        body,
        out_type=(
            jax.ShapeDtypeStruct((nclause * G * B,), jnp.float32),
            jax.ShapeDtypeStruct((nclause * G * B,), jnp.float32),
        ),
        mesh=plsc.VectorSubcoreMesh(
            core_axis_name="c", subcore_axis_name="s",
            num_cores=NC, num_subcores=NS,
        ),
        scratch_types=[
            pltpu.VMEM((RING, IDX_ROWS, 128), jnp.int32),
            pltpu.VMEM((RING, IDX_ROWS, 128, B), jnp.float32),
            pltpu.VMEM((RING, NG * B), jnp.float32),
            pltpu.VMEM((RING, NG * B), jnp.float32),
        ] + [pltpu.SemaphoreType.DMA] * 12,
        compiler_params=pltpu.CompilerParams(use_tc_tiling_on_sc=False),
    )


_sc_clause = _make_sc(1)
_sc_all = _make_sc(C)


def _tc_body1(R_ref, *refs):
    ms = refs[0:C]
    ss = refs[C:2 * C]
    out_ref = refs[2 * C]
    us = []
    M = None
    for c in range(C):
        t = ms[c][:] + GAMMA * jnp.log(ss[c][:])
        mx = jnp.max(t)
        r = t / jnp.maximum(mx, 1.0)
        Rc = R_ref[:]
        mm = jnp.maximum(Rc, r)
        u = mm + GAMMA * jnp.log(
            jnp.exp((Rc - mm) * INVG) + jnp.exp((r - mm) * INVG)
        )
        us.append(u)
        uM = jnp.max(u)
        M = uM if M is None else jnp.maximum(M, uM)
    scale = 1.0 / jnp.maximum(M, 1.0)
    for c in range(C):
        out_ref[c] = us[c] * scale


def _tc_body2(R_ref, m_ref, s_ref, out_ref):
    us = []
    M = None
    for c in range(C):
        t = m_ref[c] + GAMMA * jnp.log(s_ref[c])
        mx = jnp.max(t)
        r = t / jnp.maximum(mx, 1.0)
        Rc = R_ref[c]
        mm = jnp.maximum(Rc, r)
        u = mm + GAMMA * jnp.log(
            jnp.exp((Rc - mm) * INVG) + jnp.exp((r - mm) * INVG)
        )
        us.append(u)
        uM = jnp.max(u)
        M = uM if M is None else jnp.maximum(M, uM)
    scale = 1.0 / jnp.maximum(M, 1.0)
    for c in range(C):
        out_ref[c] = us[c] * scale


_tc_combine1 = pl.pallas_call(
    _tc_body1,
    out_shape=jax.ShapeDtypeStruct((C, G * B), jnp.float32),
)
_tc_combine2 = pl.pallas_call(
    _tc_body2,
    out_shape=jax.ShapeDtypeStruct((C, G * B), jnp.float32),
)


def kernel(x, I):
    xT = x.T
    iflats = lax.optimization_barrier(
        [I[c].reshape(G * S * L // 128, 128) for c in range(C)])
    parts = [_sc_clause(xT, iflats[c]) for c in range(C)]
    ms = [p[0] for p in parts]
    ss = [p[1] for p in parts]
    Rt = _tc_combine1(xT.reshape(G * B), *ms, *ss)
    iadj = jnp.stack([iflats[c] + c * G for c in range(C)]).reshape(-1, 128)
    for _ in range(INFER_STEP - 1):
        m, s = _sc_all(Rt.reshape(C * G, B), iadj)
        Rt = _tc_combine2(Rt, m.reshape(C, G * B), s.reshape(C, G * B))
    return Rt.reshape(C, G, B).transpose(0, 2, 1)
